# baseline (device time: 98091 ns/iter reference)
import jax
import jax.numpy as jnp
from jax import lax
from jax.experimental import pallas as pl
from jax.experimental.pallas import tpu as pltpu

T_LOC = 512
D = 1024
F = 2048
E_LOC = 4
E = 8


def _top2_weights(g):
    m1 = jnp.max(g, axis=1, keepdims=True)
    is1 = g == m1
    gneg = jnp.where(is1, -jnp.inf, g)
    m2 = jnp.max(gneg, axis=1, keepdims=True)
    is2 = gneg == m2
    e2 = jnp.exp(m2 - m1)
    w1 = 1.0 / (1.0 + e2)
    w2 = e2 / (1.0 + e2)
    return is1 * w1 + is2 * w2


def kernel(x, router, W1, W2):
    def body(x_ref, r_ref, w1_ref, w2_ref, out_ref,
             xs_bf, xp_bf, rrecv, gsend, grecv, psend, precv,
             w1s, w2s, w1b, w2b, sems):
        my_x = lax.axis_index("x")
        my_y = lax.axis_index("y")
        my_z = lax.axis_index("z")
        peer = (1 - my_x, my_y, my_z)

        barrier = pltpu.get_barrier_semaphore()
        pl.semaphore_signal(barrier, inc=1, device_id=peer,
                            device_id_type=pl.DeviceIdType.MESH)
        pl.semaphore_wait(barrier, 1)

        xs_bf[...] = x_ref[...].astype(jnp.bfloat16)
        rd_x = pltpu.make_async_remote_copy(
            src_ref=xs_bf, dst_ref=xp_bf,
            send_sem=sems.at[0], recv_sem=sems.at[1],
            device_id=peer, device_id_type=pl.DeviceIdType.MESH)
        rd_x.start()
        rd_r = pltpu.make_async_remote_copy(
            src_ref=r_ref, dst_ref=rrecv,
            send_sem=sems.at[2], recv_sem=sems.at[3],
            device_id=peer, device_id_type=pl.DeviceIdType.MESH)
        rd_r.start()

        def w_dma(e):
            return (pltpu.make_async_copy(w1_ref.at[e], w1s, sems.at[8]),
                    pltpu.make_async_copy(w2_ref.at[e], w2s, sems.at[9]))
        for cp in w_dma(0):
            cp.start()

        rd_r.wait()
        xf = x_ref[...]
        dot_f32 = lambda a, b: lax.dot_general(
            a, b, (((1,), (0,)), ((), ())), precision=lax.Precision.HIGHEST)
        g_loc = dot_f32(xf, r_ref[...])
        g_pe = dot_f32(xf, rrecv[...])
        g_own = jnp.where(my_x == 0,
                          jnp.concatenate([g_loc, g_pe], axis=1),
                          jnp.concatenate([g_pe, g_loc], axis=1))
        gsend[...] = g_own
        rd_g = pltpu.make_async_remote_copy(
            src_ref=gsend, dst_ref=grecv,
            send_sem=sems.at[4], recv_sem=sems.at[5],
            device_id=peer, device_id_type=pl.DeviceIdType.MESH)
        rd_g.start()

        wd_own = _top2_weights(g_own)
        wo = jnp.where(my_x == 0, wd_own[:, :E_LOC], wd_own[:, E_LOC:])

        def ffn(xblk, w1e, w2e):
            o = jnp.zeros((T_LOC, D), jnp.float32)
            for f0 in (0, F // 2):
                h = jnp.dot(xblk, w1e[:, f0:f0 + F // 2],
                            preferred_element_type=jnp.float32)
                h = jnp.maximum(h, 0.0).astype(jnp.bfloat16)
                o = o + jnp.dot(h, w2e[f0:f0 + F // 2, :],
                                preferred_element_type=jnp.float32)
            return o

        acc_o = jnp.zeros((T_LOC, D), jnp.float32)
        acc_p = jnp.zeros((T_LOC, D), jnp.float32)
        wp = None
        for e in range(E_LOC):
            slot = e % 2
            for cp in w_dma(e):
                cp.wait()
            w1b[slot] = w1s[...].astype(jnp.bfloat16)
            w2b[slot] = w2s[...].astype(jnp.bfloat16)
            if e + 1 < E_LOC:
                for cp in w_dma(e + 1):
                    cp.start()
            acc_o = acc_o + ffn(xs_bf[...], w1b[slot], w2b[slot]) \
                * wo[:, e][:, None]
            if e == 0:
                rd_x.wait()
                rd_g.wait()
                wd_peer = _top2_weights(grecv[...])
                wp = jnp.where(my_x == 0, wd_peer[:, :E_LOC],
                               wd_peer[:, E_LOC:])
            acc_p = acc_p + ffn(xp_bf[...], w1b[slot], w2b[slot]) \
                * wp[:, e][:, None]

        psend[...] = acc_p.astype(jnp.bfloat16)
        rd_p = pltpu.make_async_remote_copy(
            src_ref=psend, dst_ref=precv,
            send_sem=sems.at[6], recv_sem=sems.at[7],
            device_id=peer, device_id_type=pl.DeviceIdType.MESH)
        rd_p.start()
        rd_p.wait()
        out_ref[...] = acc_o + precv[...].astype(jnp.float32)

    return pl.pallas_call(
        body,
        out_shape=jax.ShapeDtypeStruct((T_LOC, D), jnp.float32),
        in_specs=[
            pl.BlockSpec(memory_space=pltpu.VMEM),
            pl.BlockSpec(memory_space=pltpu.VMEM),
            pl.BlockSpec(memory_space=pltpu.MemorySpace.HBM),
            pl.BlockSpec(memory_space=pltpu.MemorySpace.HBM),
        ],
        out_specs=pl.BlockSpec(memory_space=pltpu.VMEM),
        scratch_shapes=[
            pltpu.VMEM((T_LOC, D), jnp.bfloat16),
            pltpu.VMEM((T_LOC, D), jnp.bfloat16),
            pltpu.VMEM((D, E_LOC), jnp.float32),
            pltpu.VMEM((T_LOC, E), jnp.float32),
            pltpu.VMEM((T_LOC, E), jnp.float32),
            pltpu.VMEM((T_LOC, D), jnp.bfloat16),
            pltpu.VMEM((T_LOC, D), jnp.bfloat16),
            pltpu.VMEM((D, F), jnp.float32),
            pltpu.VMEM((F, D), jnp.float32),
            pltpu.VMEM((2, D, F), jnp.bfloat16),
            pltpu.VMEM((2, F, D), jnp.bfloat16),
            pltpu.SemaphoreType.DMA((10,)),
        ],
        compiler_params=pltpu.CompilerParams(
            collective_id=0, vmem_limit_bytes=56 * 1024 * 1024),
    )(x, router, W1, W2)


# device time: 87643 ns/iter; 1.1192x vs baseline; 1.1192x over previous
import jax
import jax.numpy as jnp
from jax import lax
from jax.experimental import pallas as pl
from jax.experimental.pallas import tpu as pltpu

T_LOC = 512
D = 1024
F = 2048
E_LOC = 4
E = 8


def _top2_weights(g):
    m1 = jnp.max(g, axis=1, keepdims=True)
    is1 = g == m1
    gneg = jnp.where(is1, -jnp.inf, g)
    m2 = jnp.max(gneg, axis=1, keepdims=True)
    is2 = gneg == m2
    e2 = jnp.exp(m2 - m1)
    w1 = 1.0 / (1.0 + e2)
    w2 = e2 / (1.0 + e2)
    return is1 * w1 + is2 * w2


def kernel(x, router, W1, W2):
    def body(x_ref, r_ref, w1_ref, w2_ref, out_ref,
             xs_bf, xp_bf, rrecv, gsend, grecv, psend, precv,
             w1s, w2s, sems):
        my_x = lax.axis_index("x")
        my_y = lax.axis_index("y")
        my_z = lax.axis_index("z")
        peer = (1 - my_x, my_y, my_z)

        barrier = pltpu.get_barrier_semaphore()
        pl.semaphore_signal(barrier, inc=1, device_id=peer,
                            device_id_type=pl.DeviceIdType.MESH)
        pl.semaphore_wait(barrier, 1)

        xs_bf[...] = x_ref[...].astype(jnp.bfloat16)
        rd_x = pltpu.make_async_remote_copy(
            src_ref=xs_bf, dst_ref=xp_bf,
            send_sem=sems.at[0], recv_sem=sems.at[1],
            device_id=peer, device_id_type=pl.DeviceIdType.MESH)
        rd_x.start()
        rd_r = pltpu.make_async_remote_copy(
            src_ref=r_ref, dst_ref=rrecv,
            send_sem=sems.at[2], recv_sem=sems.at[3],
            device_id=peer, device_id_type=pl.DeviceIdType.MESH)
        rd_r.start()

        def w1_dma(e):
            slot = e % 2
            return pltpu.make_async_copy(w1_ref.at[e], w1s.at[slot],
                                         sems.at[8 + slot])

        def w2_dma(e):
            return pltpu.make_async_copy(w2_ref.at[e], w2s, sems.at[10])

        w1_dma(0).start()
        w1_dma(1).start()
        w2_dma(0).start()

        rd_r.wait()
        xf = x_ref[...]
        dot_f32 = lambda a, b: lax.dot_general(
            a, b, (((1,), (0,)), ((), ())), precision=lax.Precision.HIGHEST)
        g_loc = dot_f32(xf, r_ref[...])
        g_pe = dot_f32(xf, rrecv[...])
        g_own = jnp.where(my_x == 0,
                          jnp.concatenate([g_loc, g_pe], axis=1),
                          jnp.concatenate([g_pe, g_loc], axis=1))
        gsend[...] = g_own
        rd_g = pltpu.make_async_remote_copy(
            src_ref=gsend, dst_ref=grecv,
            send_sem=sems.at[4], recv_sem=sems.at[5],
            device_id=peer, device_id_type=pl.DeviceIdType.MESH)
        rd_g.start()

        wd_own = _top2_weights(g_own)
        wo = jnp.where(my_x == 0, wd_own[:, :E_LOC], wd_own[:, E_LOC:])

        def ffn(xblk, w1e, w2e):
            o = jnp.zeros((T_LOC, D), jnp.float32)
            for f0 in (0, F // 2):
                h = jnp.dot(xblk, w1e[:, f0:f0 + F // 2],
                            preferred_element_type=jnp.float32)
                h = jnp.maximum(h, 0.0).astype(jnp.bfloat16)
                o = o + jnp.dot(h, w2e[f0:f0 + F // 2, :],
                                preferred_element_type=jnp.float32)
            return o

        acc_o = jnp.zeros((T_LOC, D), jnp.float32)
        acc_p = jnp.zeros((T_LOC, D), jnp.float32)
        wp = None
        own_last = None
        for e in range(E_LOC):
            slot = e % 2
            w1_dma(e).wait()
            w2_dma(e).wait()
            w1e = w1s[slot].astype(jnp.bfloat16)
            w2e = w2s[...].astype(jnp.bfloat16)
            if e + 1 < E_LOC:
                w2_dma(e + 1).start()
            if e + 2 < E_LOC:
                w1_dma(e + 2).start()
            if e == 0:
                acc_o = acc_o + ffn(xs_bf[...], w1e, w2e) * wo[:, e][:, None]
                rd_x.wait()
                rd_g.wait()
                wd_peer = _top2_weights(grecv[...])
                wp = jnp.where(my_x == 0, wd_peer[:, :E_LOC],
                               wd_peer[:, E_LOC:])
                acc_p = acc_p + ffn(xp_bf[...], w1e, w2e) * wp[:, e][:, None]
            elif e < E_LOC - 1:
                acc_p = acc_p + ffn(xp_bf[...], w1e, w2e) * wp[:, e][:, None]
                acc_o = acc_o + ffn(xs_bf[...], w1e, w2e) * wo[:, e][:, None]
            else:
                acc_p = acc_p + ffn(xp_bf[...], w1e, w2e) * wp[:, e][:, None]
                own_last = (w1e, w2e)

        psend[...] = acc_p.astype(jnp.bfloat16)
        rd_p = pltpu.make_async_remote_copy(
            src_ref=psend, dst_ref=precv,
            send_sem=sems.at[6], recv_sem=sems.at[7],
            device_id=peer, device_id_type=pl.DeviceIdType.MESH)
        rd_p.start()
        w1e, w2e = own_last
        acc_o = acc_o + ffn(xs_bf[...], w1e, w2e) * wo[:, E_LOC - 1][:, None]
        rd_p.wait()
        out_ref[...] = acc_o + precv[...].astype(jnp.float32)

    return pl.pallas_call(
        body,
        out_shape=jax.ShapeDtypeStruct((T_LOC, D), jnp.float32),
        in_specs=[
            pl.BlockSpec(memory_space=pltpu.VMEM),
            pl.BlockSpec(memory_space=pltpu.VMEM),
            pl.BlockSpec(memory_space=pltpu.MemorySpace.HBM),
            pl.BlockSpec(memory_space=pltpu.MemorySpace.HBM),
        ],
        out_specs=pl.BlockSpec(memory_space=pltpu.VMEM),
        scratch_shapes=[
            pltpu.VMEM((T_LOC, D), jnp.bfloat16),
            pltpu.VMEM((T_LOC, D), jnp.bfloat16),
            pltpu.VMEM((D, E_LOC), jnp.float32),
            pltpu.VMEM((T_LOC, E), jnp.float32),
            pltpu.VMEM((T_LOC, E), jnp.float32),
            pltpu.VMEM((T_LOC, D), jnp.bfloat16),
            pltpu.VMEM((T_LOC, D), jnp.bfloat16),
            pltpu.VMEM((2, D, F), jnp.float32),
            pltpu.VMEM((F, D), jnp.float32),
            pltpu.SemaphoreType.DMA((12,)),
        ],
        compiler_params=pltpu.CompilerParams(
            collective_id=0, vmem_limit_bytes=56 * 1024 * 1024),
    )(x, router, W1, W2)
